# baseline (speedup 1.0000x reference)
import functools

import jax
import jax.numpy as jnp
from jax import lax
from jax.experimental import pallas as pl
from jax.experimental.pallas import tpu as pltpu
from jax.experimental.pallas import tpu_sc as plsc

N = 32768
D = 768
BLK = 2048
NBLK = N // BLK

NW = 32
LANES = 16
ROWS = N // LANES
WROWS = ROWS // NW
_W1_OFF = 0
_B1_OFF = 32
_W2_OFF = 64
_B2_OFF = 576
_W3_OFF = 592
_B3_OFF = 640
_P_LEN = 704


def _lane_max(v):
    idx = lax.iota(jnp.int32, LANES)
    for s in (8, 4, 2, 1):
        perm = jnp.bitwise_xor(idx, s)
        v = jnp.maximum(v, v.at[perm].get(mode="promise_in_bounds"))
    return v


def _sc_router_body(u_hbm, p_hbm, mask_hbm, wf_hbm, u_v, p_v, mask_v, wf_v):
    w = lax.axis_index("s") * 2 + lax.axis_index("c")
    pltpu.sync_copy(u_hbm, u_v)
    pltpu.sync_copy(p_hbm, p_v)

    def P(idx):
        return p_v[pl.ds((idx // LANES) * LANES, LANES)][idx % LANES]

    def mm_body(it, carry):
        mn, mx = carry
        for r in range(8):
            v = u_v[pl.ds((it * 8 + r) * LANES, LANES)]
            mn = jnp.minimum(mn, v)
            mx = jnp.maximum(mx, v)
        return mn, mx

    v0 = u_v[pl.ds(0, LANES)]
    mn, mx = lax.fori_loop(0, ROWS // 8, mm_body, (v0, v0))
    gmin = -_lane_max(-mn)[0]
    gmax = _lane_max(mx)[0]
    denom = gmax - gmin + 1e-8

    base = w * WROWS

    def row_body(i, bits):
        b0, b1, b2 = bits
        v = u_v[pl.ds((base + i) * LANES, LANES)]
        un = (v - gmin) / denom
        a = [jnp.full((LANES,), P(_B2_OFF + k)) for k in range(16)]
        for j in range(32):
            hj = jnp.maximum(un * P(_W1_OFF + j) + P(_B1_OFF + j), 0.0)
            for k in range(16):
                a[k] = a[k] + hj * P(_W2_OFF + j * 16 + k)
        l0 = jnp.full((LANES,), P(_B3_OFF + 0))
        l1 = jnp.full((LANES,), P(_B3_OFF + 1))
        l2 = jnp.full((LANES,), P(_B3_OFF + 2))
        for k in range(16):
            hk = jnp.maximum(a[k], 0.0)
            l0 = l0 + hk * P(_W3_OFF + k * 3 + 0)
            l1 = l1 + hk * P(_W3_OFF + k * 3 + 1)
            l2 = l2 + hk * P(_W3_OFF + k * 3 + 2)
        d = jnp.where((l1 > l0) & (l1 >= l2), 1.0,
                      jnp.where((l2 > l0) & (l2 > l1), 2.0, 0.0))
        mask_v[pl.ds(i * LANES, LANES)] = d
        one = jnp.ones((LANES,), jnp.float32)
        zero = jnp.zeros((LANES,), jnp.float32)
        b0 = jnp.maximum(b0, jnp.where(d == 0.0, one, zero))
        b1 = jnp.maximum(b1, jnp.where(d == 1.0, one, zero))
        b2 = jnp.maximum(b2, jnp.where(d == 2.0, one, zero))
        return b0, b1, b2

    z = jnp.zeros((LANES,), jnp.float32)
    b0, b1, b2 = lax.fori_loop(0, WROWS, row_body, (z, z, z))
    wflag = (_lane_max(b0)[0] + 2.0 * _lane_max(b1)[0]
             + 4.0 * _lane_max(b2)[0]).astype(jnp.int32)
    wf_v[...] = jnp.full((LANES,), wflag, jnp.int32)
    pltpu.sync_copy(mask_v, mask_hbm.at[pl.ds(base * LANES, WROWS * LANES)])
    pltpu.sync_copy(wf_v, wf_hbm.at[pl.ds(w * LANES, LANES)])


def _fuse_body(w0_ref, b0_ref, w1_ref, b1_ref, w2_ref, b2_ref, w3_ref, b3_ref,
               wc1_ref, bc1_ref, wc2_ref, bc2_ref):
    w01 = jnp.dot(w0_ref[...], w1_ref[...], preferred_element_type=jnp.float32)
    wc1_ref[...] = w01
    bc1 = jnp.dot(b0_ref[...], w1_ref[...], preferred_element_type=jnp.float32) + b1_ref[...]
    bc1_ref[...] = bc1
    w012 = jnp.dot(w01, w2_ref[...], preferred_element_type=jnp.float32)
    wc2_ref[...] = jnp.dot(w012, w3_ref[...], preferred_element_type=jnp.float32)
    bc2 = jnp.dot(bc1, w2_ref[...], preferred_element_type=jnp.float32) + b2_ref[...]
    bc2_ref[...] = jnp.dot(bc2, w3_ref[...], preferred_element_type=jnp.float32) + b3_ref[...]


def _chain_body(flags_ref, x_ref, m_ref,
                w0_ref, b0_ref, wc1_ref, bc1_ref, wc2_ref, bc2_ref,
                out_ref):
    i = pl.program_id(0)
    f = flags_ref[i]
    x = x_ref[...]
    m = m_ref[...]

    @pl.when((f & 1) != 0)
    def _():
        out_ref[...] = jnp.dot(x, w0_ref[...], preferred_element_type=jnp.float32) + b0_ref[...]

    @pl.when((f & 2) != 0)
    def _():
        s1 = jnp.dot(x, wc1_ref[...], preferred_element_type=jnp.float32) + bc1_ref[...]

        @pl.when((f & 1) != 0)
        def _():
            out_ref[...] = jnp.where(m == 1.0, s1, out_ref[...])

        @pl.when((f & 1) == 0)
        def _():
            out_ref[...] = s1

    @pl.when((f & 4) != 0)
    def _():
        s2 = jnp.dot(x, wc2_ref[...], preferred_element_type=jnp.float32) + bc2_ref[...]

        @pl.when((f & 3) != 0)
        def _():
            out_ref[...] = jnp.where(m == 2.0, s2, out_ref[...])

        @pl.when((f & 3) == 0)
        def _():
            out_ref[...] = s2


_INTERPRET = False


def _full(shape):
    return pl.BlockSpec(shape, lambda i, flags: (0, 0))


def kernel(x, current_uncertainty, rW1, rb1, rW2, rb2, rW3, rb3,
           W0, b0, W1, b1, W2, b2, W3, b3):
    params = jnp.concatenate([
        rW1.ravel(), rb1, rW2.ravel(), rb2, rW3.ravel(), rb3,
        jnp.zeros((_P_LEN - 643,), jnp.float32),
    ])

    sc_router = pl.kernel(
        _sc_router_body,
        out_type=(jax.ShapeDtypeStruct((N,), jnp.float32),
                  jax.ShapeDtypeStruct((NW * LANES,), jnp.int32)),
        mesh=plsc.VectorSubcoreMesh(core_axis_name="c", subcore_axis_name="s"),
        scratch_types=[
            pltpu.VMEM((N,), jnp.float32),
            pltpu.VMEM((_P_LEN,), jnp.float32),
            pltpu.VMEM((WROWS * LANES,), jnp.float32),
            pltpu.VMEM((LANES,), jnp.int32),
        ],
        interpret=_INTERPRET,
    )
    mask, wflags = sc_router(current_uncertainty, params)

    wf = wflags.reshape(NW, LANES)[:, 0].reshape(NBLK, NW // NBLK)
    flags = functools.reduce(jnp.bitwise_or,
                             [wf[:, g] for g in range(NW // NBLK)])

    wc1, bc1, wc2, bc2 = pl.pallas_call(
        _fuse_body,
        out_shape=(jax.ShapeDtypeStruct((D, D), jnp.float32),
                   jax.ShapeDtypeStruct((1, D), jnp.float32),
                   jax.ShapeDtypeStruct((D, D), jnp.float32),
                   jax.ShapeDtypeStruct((1, D), jnp.float32)),
        interpret=_INTERPRET,
    )(W0, b0.reshape(1, D), W1, b1.reshape(1, D),
      W2, b2.reshape(1, D), W3, b3.reshape(1, D))

    grid_spec = pltpu.PrefetchScalarGridSpec(
        num_scalar_prefetch=1,
        grid=(NBLK,),
        in_specs=[
            pl.BlockSpec((BLK, D), lambda i, flags: (i, 0)),
            pl.BlockSpec((BLK, 1), lambda i, flags: (i, 0)),
            _full((D, D)), _full((1, D)),
            _full((D, D)), _full((1, D)),
            _full((D, D)), _full((1, D)),
        ],
        out_specs=pl.BlockSpec((BLK, D), lambda i, flags: (i, 0)),
    )
    out = pl.pallas_call(
        _chain_body,
        grid_spec=grid_spec,
        out_shape=jax.ShapeDtypeStruct((N, D), jnp.float32),
        interpret=_INTERPRET,
    )(flags, x, mask.reshape(N, 1),
      W0, b0.reshape(1, D), wc1, bc1, wc2, bc2)
    return out, mask


# speedup vs baseline: 1.0511x; 1.0511x over previous
import functools

import jax
import jax.numpy as jnp
from jax import lax
from jax.experimental import pallas as pl
from jax.experimental.pallas import tpu as pltpu
from jax.experimental.pallas import tpu_sc as plsc

N = 32768
D = 768
BLK = 2048
NBLK = N // BLK

NW = 32
LANES = 16
ROWS = N // LANES
WROWS = ROWS // NW
_W1_OFF = 0
_B1_OFF = 32
_W2_OFF = 64
_B2_OFF = 576
_W3_OFF = 592
_B3_OFF = 640
_P_LEN = 704


def _lane_max(v):
    idx = lax.iota(jnp.int32, LANES)
    for s in (8, 4, 2, 1):
        perm = jnp.bitwise_xor(idx, s)
        v = jnp.maximum(v, v.at[perm].get(mode="promise_in_bounds"))
    return v


def _sc_router_body(u_hbm, p_hbm, mask_hbm, wf_hbm,
                    u_v, uscan_v, p_v, mask_v, wf_v, mnmx_v, shrb_v, sh_spm):
    c = lax.axis_index("c")
    s = lax.axis_index("s")
    w = s * 2 + c
    SCAN_ROWS = ROWS // LANES
    pltpu.sync_copy(u_hbm.at[pl.ds(w * WROWS * LANES, WROWS * LANES)], u_v)
    pltpu.sync_copy(u_hbm.at[pl.ds(s * SCAN_ROWS * LANES, SCAN_ROWS * LANES)],
                    uscan_v)
    pltpu.sync_copy(p_hbm, p_v)

    def P(idx):
        return p_v[pl.ds((idx // LANES) * LANES, LANES)][idx % LANES]

    def mm_body(it, carry):
        mn, mx = carry
        for r in range(8):
            v = uscan_v[pl.ds((it * 8 + r) * LANES, LANES)]
            mn = jnp.minimum(mn, v)
            mx = jnp.maximum(mx, v)
        return mn, mx

    v0 = uscan_v[pl.ds(0, LANES)]
    mn, mx = lax.fori_loop(0, SCAN_ROWS // 8, mm_body, (v0, v0))
    mnmx_v[pl.ds(0, LANES)] = mn
    mnmx_v[pl.ds(LANES, LANES)] = mx
    pltpu.sync_copy(mnmx_v, sh_spm.at[pl.ds(s * 2 * LANES, 2 * LANES)])
    plsc.subcore_barrier()
    pltpu.sync_copy(sh_spm, shrb_v)
    for r in range(1, LANES):
        mn = jnp.minimum(mn, shrb_v[pl.ds(r * 2 * LANES, LANES)])
        mx = jnp.maximum(mx, shrb_v[pl.ds(r * 2 * LANES + LANES, LANES)])
    mn = jnp.minimum(mn, shrb_v[pl.ds(0, LANES)])
    mx = jnp.maximum(mx, shrb_v[pl.ds(LANES, LANES)])
    gmin = -_lane_max(-mn)[0]
    gmax = _lane_max(mx)[0]
    denom = gmax - gmin + 1e-8

    def row_body(i, bits):
        b0, b1, b2 = bits
        v = u_v[pl.ds(i * LANES, LANES)]
        un = (v - gmin) / denom
        a = [jnp.full((LANES,), P(_B2_OFF + k)) for k in range(16)]
        for j in range(32):
            hj = jnp.maximum(un * P(_W1_OFF + j) + P(_B1_OFF + j), 0.0)
            for k in range(16):
                a[k] = a[k] + hj * P(_W2_OFF + j * 16 + k)
        l0 = jnp.full((LANES,), P(_B3_OFF + 0))
        l1 = jnp.full((LANES,), P(_B3_OFF + 1))
        l2 = jnp.full((LANES,), P(_B3_OFF + 2))
        for k in range(16):
            hk = jnp.maximum(a[k], 0.0)
            l0 = l0 + hk * P(_W3_OFF + k * 3 + 0)
            l1 = l1 + hk * P(_W3_OFF + k * 3 + 1)
            l2 = l2 + hk * P(_W3_OFF + k * 3 + 2)
        d = jnp.where((l1 > l0) & (l1 >= l2), 1.0,
                      jnp.where((l2 > l0) & (l2 > l1), 2.0, 0.0))
        mask_v[pl.ds(i * LANES, LANES)] = d
        one = jnp.ones((LANES,), jnp.float32)
        zero = jnp.zeros((LANES,), jnp.float32)
        b0 = jnp.maximum(b0, jnp.where(d == 0.0, one, zero))
        b1 = jnp.maximum(b1, jnp.where(d == 1.0, one, zero))
        b2 = jnp.maximum(b2, jnp.where(d == 2.0, one, zero))
        return b0, b1, b2

    z = jnp.zeros((LANES,), jnp.float32)
    b0, b1, b2 = lax.fori_loop(0, WROWS, row_body, (z, z, z))
    wflag = (_lane_max(b0)[0] + 2.0 * _lane_max(b1)[0]
             + 4.0 * _lane_max(b2)[0]).astype(jnp.int32)
    wf_v[...] = jnp.full((LANES,), wflag, jnp.int32)
    pltpu.sync_copy(mask_v, mask_hbm.at[pl.ds(w * WROWS * LANES, WROWS * LANES)])
    pltpu.sync_copy(wf_v, wf_hbm.at[pl.ds(w * LANES, LANES)])


def _fuse_body(w0_ref, b0_ref, w1_ref, b1_ref, w2_ref, b2_ref, w3_ref, b3_ref,
               wc1_ref, bc1_ref, wc2_ref, bc2_ref):
    w01 = jnp.dot(w0_ref[...], w1_ref[...], preferred_element_type=jnp.float32)
    wc1_ref[...] = w01
    bc1 = jnp.dot(b0_ref[...], w1_ref[...], preferred_element_type=jnp.float32) + b1_ref[...]
    bc1_ref[...] = bc1
    w012 = jnp.dot(w01, w2_ref[...], preferred_element_type=jnp.float32)
    wc2_ref[...] = jnp.dot(w012, w3_ref[...], preferred_element_type=jnp.float32)
    bc2 = jnp.dot(bc1, w2_ref[...], preferred_element_type=jnp.float32) + b2_ref[...]
    bc2_ref[...] = jnp.dot(bc2, w3_ref[...], preferred_element_type=jnp.float32) + b3_ref[...]


def _chain_body(flags_ref, x_ref, m_ref,
                w0_ref, b0_ref, wc1_ref, bc1_ref, wc2_ref, bc2_ref,
                out_ref):
    i = pl.program_id(0)
    f = flags_ref[i]
    x = x_ref[...]
    m = m_ref[...]

    @pl.when((f & 1) != 0)
    def _():
        out_ref[...] = jnp.dot(x, w0_ref[...], preferred_element_type=jnp.float32) + b0_ref[...]

    @pl.when((f & 2) != 0)
    def _():
        s1 = jnp.dot(x, wc1_ref[...], preferred_element_type=jnp.float32) + bc1_ref[...]

        @pl.when((f & 1) != 0)
        def _():
            out_ref[...] = jnp.where(m == 1.0, s1, out_ref[...])

        @pl.when((f & 1) == 0)
        def _():
            out_ref[...] = s1

    @pl.when((f & 4) != 0)
    def _():
        s2 = jnp.dot(x, wc2_ref[...], preferred_element_type=jnp.float32) + bc2_ref[...]

        @pl.when((f & 3) != 0)
        def _():
            out_ref[...] = jnp.where(m == 2.0, s2, out_ref[...])

        @pl.when((f & 3) == 0)
        def _():
            out_ref[...] = s2


_INTERPRET = False


def _full(shape):
    return pl.BlockSpec(shape, lambda i, flags: (0, 0))


def kernel(x, current_uncertainty, rW1, rb1, rW2, rb2, rW3, rb3,
           W0, b0, W1, b1, W2, b2, W3, b3):
    params = jnp.concatenate([
        rW1.ravel(), rb1, rW2.ravel(), rb2, rW3.ravel(), rb3,
        jnp.zeros((_P_LEN - 643,), jnp.float32),
    ])

    sc_router = pl.kernel(
        _sc_router_body,
        out_type=(jax.ShapeDtypeStruct((N,), jnp.float32),
                  jax.ShapeDtypeStruct((NW * LANES,), jnp.int32)),
        mesh=plsc.VectorSubcoreMesh(core_axis_name="c", subcore_axis_name="s"),
        scratch_types=[
            pltpu.VMEM((WROWS * LANES,), jnp.float32),
            pltpu.VMEM((N // LANES,), jnp.float32),
            pltpu.VMEM((_P_LEN,), jnp.float32),
            pltpu.VMEM((WROWS * LANES,), jnp.float32),
            pltpu.VMEM((LANES,), jnp.int32),
            pltpu.VMEM((2 * LANES,), jnp.float32),
            pltpu.VMEM((2 * LANES * LANES,), jnp.float32),
            pltpu.VMEM_SHARED((2 * LANES * LANES,), jnp.float32),
        ],
        interpret=_INTERPRET,
    )
    mask, wflags = sc_router(current_uncertainty, params)

    wf = wflags.reshape(NW, LANES)[:, 0].reshape(NBLK, NW // NBLK)
    flags = functools.reduce(jnp.bitwise_or,
                             [wf[:, g] for g in range(NW // NBLK)])

    wc1, bc1, wc2, bc2 = pl.pallas_call(
        _fuse_body,
        out_shape=(jax.ShapeDtypeStruct((D, D), jnp.float32),
                   jax.ShapeDtypeStruct((1, D), jnp.float32),
                   jax.ShapeDtypeStruct((D, D), jnp.float32),
                   jax.ShapeDtypeStruct((1, D), jnp.float32)),
        interpret=_INTERPRET,
    )(W0, b0.reshape(1, D), W1, b1.reshape(1, D),
      W2, b2.reshape(1, D), W3, b3.reshape(1, D))

    grid_spec = pltpu.PrefetchScalarGridSpec(
        num_scalar_prefetch=1,
        grid=(NBLK,),
        in_specs=[
            pl.BlockSpec((BLK, D), lambda i, flags: (i, 0)),
            pl.BlockSpec((BLK, 1), lambda i, flags: (i, 0)),
            _full((D, D)), _full((1, D)),
            _full((D, D)), _full((1, D)),
            _full((D, D)), _full((1, D)),
        ],
        out_specs=pl.BlockSpec((BLK, D), lambda i, flags: (i, 0)),
    )
    out = pl.pallas_call(
        _chain_body,
        grid_spec=grid_spec,
        out_shape=jax.ShapeDtypeStruct((N, D), jnp.float32),
        interpret=_INTERPRET,
    )(flags, x, mask.reshape(N, 1),
      W0, b0.reshape(1, D), wc1, bc1, wc2, bc2)
    return out, mask


# speedup vs baseline: 1.0935x; 1.0403x over previous
import functools

import jax
import jax.numpy as jnp
from jax import lax
from jax.experimental import pallas as pl
from jax.experimental.pallas import tpu as pltpu
from jax.experimental.pallas import tpu_sc as plsc

N = 32768
D = 768
BLK = 2048
NBLK = N // BLK

NW = 32
LANES = 16
ROWS = N // LANES
WROWS = ROWS // NW
_W1_OFF = 0
_B1_OFF = 32
_W2_OFF = 64
_B2_OFF = 576
_W3_OFF = 592
_B3_OFF = 640
_P_LEN = 704


def _lane_max(v):
    idx = lax.iota(jnp.int32, LANES)
    for s in (8, 4, 2, 1):
        perm = jnp.bitwise_xor(idx, s)
        v = jnp.maximum(v, v.at[perm].get(mode="promise_in_bounds"))
    return v


def _sc_router_body(u_hbm, p_hbm, mask_hbm, wf_hbm,
                    u_v, uscan_v, p_v, mask_v, wf_v, mnmx_v, shrb_v, sh_spm,
                    w1b_v, b1b_v, w2b_v, b2b_v, w3b_v, b3b_v):
    c = lax.axis_index("c")
    s = lax.axis_index("s")
    w = s * 2 + c
    SCAN_ROWS = ROWS // LANES
    pltpu.sync_copy(u_hbm.at[pl.ds(w * WROWS * LANES, WROWS * LANES)], u_v)
    pltpu.sync_copy(u_hbm.at[pl.ds(s * SCAN_ROWS * LANES, SCAN_ROWS * LANES)],
                    uscan_v)
    pltpu.sync_copy(p_hbm, p_v)

    def P(idx):
        return p_v[pl.ds((idx // LANES) * LANES, LANES)][idx % LANES]

    def mm_body(it, carry):
        mn, mx = carry
        for r in range(8):
            v = uscan_v[pl.ds((it * 8 + r) * LANES, LANES)]
            mn = jnp.minimum(mn, v)
            mx = jnp.maximum(mx, v)
        return mn, mx

    v0 = uscan_v[pl.ds(0, LANES)]
    mn, mx = lax.fori_loop(0, SCAN_ROWS // 8, mm_body, (v0, v0))
    mnmx_v[pl.ds(0, LANES)] = mn
    mnmx_v[pl.ds(LANES, LANES)] = mx
    pltpu.sync_copy(mnmx_v, sh_spm.at[pl.ds(s * 2 * LANES, 2 * LANES)])
    plsc.subcore_barrier()
    pltpu.sync_copy(sh_spm, shrb_v)
    for r in range(1, LANES):
        mn = jnp.minimum(mn, shrb_v[pl.ds(r * 2 * LANES, LANES)])
        mx = jnp.maximum(mx, shrb_v[pl.ds(r * 2 * LANES + LANES, LANES)])
    mn = jnp.minimum(mn, shrb_v[pl.ds(0, LANES)])
    mx = jnp.maximum(mx, shrb_v[pl.ds(LANES, LANES)])
    gmin = -_lane_max(-mn)[0]
    gmax = _lane_max(mx)[0]
    denom = gmax - gmin + 1e-8

    def w2_build(j, _):
        row = p_v[pl.ds(_W2_OFF + j * LANES, LANES)]
        for k in range(16):
            w2b_v[pl.ds(j * 256 + k * LANES, LANES)] = jnp.full((LANES,), row[k])
        return 0

    lax.fori_loop(0, 32, w2_build, 0)
    for t in range(32):
        w1b_v[pl.ds(t * LANES, LANES)] = jnp.full((LANES,), P(_W1_OFF + t))
        b1b_v[pl.ds(t * LANES, LANES)] = jnp.full((LANES,), P(_B1_OFF + t))
    for k in range(16):
        b2b_v[pl.ds(k * LANES, LANES)] = jnp.full((LANES,), P(_B2_OFF + k))
    for t in range(48):
        w3b_v[pl.ds(t * LANES, LANES)] = jnp.full((LANES,), P(_W3_OFF + t))
    for m in range(3):
        b3b_v[pl.ds(m * LANES, LANES)] = jnp.full((LANES,), P(_B3_OFF + m))

    def row_body(i, bits):
        bb0, bb1, bb2 = bits
        vA = u_v[pl.ds((2 * i) * LANES, LANES)]
        vB = u_v[pl.ds((2 * i + 1) * LANES, LANES)]
        unA = (vA - gmin) / denom
        unB = (vB - gmin) / denom
        aA = [b2b_v[pl.ds(k * LANES, LANES)] for k in range(16)]
        aB = list(aA)
        for j in range(32):
            w1j = w1b_v[pl.ds(j * LANES, LANES)]
            b1j = b1b_v[pl.ds(j * LANES, LANES)]
            hjA = jnp.maximum(unA * w1j + b1j, 0.0)
            hjB = jnp.maximum(unB * w1j + b1j, 0.0)
            for k in range(16):
                wv = w2b_v[pl.ds(j * 256 + k * LANES, LANES)]
                aA[k] = aA[k] + hjA * wv
                aB[k] = aB[k] + hjB * wv
        lA = [b3b_v[pl.ds(m * LANES, LANES)] for m in range(3)]
        lB = list(lA)
        for k in range(16):
            hkA = jnp.maximum(aA[k], 0.0)
            hkB = jnp.maximum(aB[k], 0.0)
            for m in range(3):
                w3v = w3b_v[pl.ds((k * 3 + m) * LANES, LANES)]
                lA[m] = lA[m] + hkA * w3v
                lB[m] = lB[m] + hkB * w3v
        one = jnp.ones((LANES,), jnp.float32)
        zero = jnp.zeros((LANES,), jnp.float32)
        dA = jnp.where((lA[1] > lA[0]) & (lA[1] >= lA[2]), 1.0,
                       jnp.where((lA[2] > lA[0]) & (lA[2] > lA[1]), 2.0, 0.0))
        dB = jnp.where((lB[1] > lB[0]) & (lB[1] >= lB[2]), 1.0,
                       jnp.where((lB[2] > lB[0]) & (lB[2] > lB[1]), 2.0, 0.0))
        mask_v[pl.ds((2 * i) * LANES, LANES)] = dA
        mask_v[pl.ds((2 * i + 1) * LANES, LANES)] = dB
        for d in (dA, dB):
            bb0 = jnp.maximum(bb0, jnp.where(d == 0.0, one, zero))
            bb1 = jnp.maximum(bb1, jnp.where(d == 1.0, one, zero))
            bb2 = jnp.maximum(bb2, jnp.where(d == 2.0, one, zero))
        return bb0, bb1, bb2

    z = jnp.zeros((LANES,), jnp.float32)
    b0, b1, b2 = lax.fori_loop(0, WROWS // 2, row_body, (z, z, z))
    wflag = (_lane_max(b0)[0] + 2.0 * _lane_max(b1)[0]
             + 4.0 * _lane_max(b2)[0]).astype(jnp.int32)
    wf_v[...] = jnp.full((LANES,), wflag, jnp.int32)
    pltpu.sync_copy(mask_v, mask_hbm.at[pl.ds(w * WROWS * LANES, WROWS * LANES)])
    pltpu.sync_copy(wf_v, wf_hbm.at[pl.ds(w * LANES, LANES)])


def _fuse_body(w0_ref, b0_ref, w1_ref, b1_ref, w2_ref, b2_ref, w3_ref, b3_ref,
               wc1_ref, bc1_ref, wc2_ref, bc2_ref):
    w01 = jnp.dot(w0_ref[...], w1_ref[...], preferred_element_type=jnp.float32)
    wc1_ref[...] = w01
    bc1 = jnp.dot(b0_ref[...], w1_ref[...], preferred_element_type=jnp.float32) + b1_ref[...]
    bc1_ref[...] = bc1
    w012 = jnp.dot(w01, w2_ref[...], preferred_element_type=jnp.float32)
    wc2_ref[...] = jnp.dot(w012, w3_ref[...], preferred_element_type=jnp.float32)
    bc2 = jnp.dot(bc1, w2_ref[...], preferred_element_type=jnp.float32) + b2_ref[...]
    bc2_ref[...] = jnp.dot(bc2, w3_ref[...], preferred_element_type=jnp.float32) + b3_ref[...]


def _chain_body(flags_ref, x_ref, m_ref,
                w0_ref, b0_ref, wc1_ref, bc1_ref, wc2_ref, bc2_ref,
                out_ref):
    i = pl.program_id(0)
    f = flags_ref[i]
    x = x_ref[...]
    m = m_ref[...]

    @pl.when((f & 1) != 0)
    def _():
        out_ref[...] = jnp.dot(x, w0_ref[...], preferred_element_type=jnp.float32) + b0_ref[...]

    @pl.when((f & 2) != 0)
    def _():
        s1 = jnp.dot(x, wc1_ref[...], preferred_element_type=jnp.float32) + bc1_ref[...]

        @pl.when((f & 1) != 0)
        def _():
            out_ref[...] = jnp.where(m == 1.0, s1, out_ref[...])

        @pl.when((f & 1) == 0)
        def _():
            out_ref[...] = s1

    @pl.when((f & 4) != 0)
    def _():
        s2 = jnp.dot(x, wc2_ref[...], preferred_element_type=jnp.float32) + bc2_ref[...]

        @pl.when((f & 3) != 0)
        def _():
            out_ref[...] = jnp.where(m == 2.0, s2, out_ref[...])

        @pl.when((f & 3) == 0)
        def _():
            out_ref[...] = s2


_INTERPRET = False


def _full(shape):
    return pl.BlockSpec(shape, lambda i, flags: (0, 0))


def kernel(x, current_uncertainty, rW1, rb1, rW2, rb2, rW3, rb3,
           W0, b0, W1, b1, W2, b2, W3, b3):
    params = jnp.concatenate([
        rW1.ravel(), rb1, rW2.ravel(), rb2, rW3.ravel(), rb3,
        jnp.zeros((_P_LEN - 643,), jnp.float32),
    ])

    sc_router = pl.kernel(
        _sc_router_body,
        out_type=(jax.ShapeDtypeStruct((N,), jnp.float32),
                  jax.ShapeDtypeStruct((NW * LANES,), jnp.int32)),
        mesh=plsc.VectorSubcoreMesh(core_axis_name="c", subcore_axis_name="s"),
        scratch_types=[
            pltpu.VMEM((WROWS * LANES,), jnp.float32),
            pltpu.VMEM((N // LANES,), jnp.float32),
            pltpu.VMEM((_P_LEN,), jnp.float32),
            pltpu.VMEM((WROWS * LANES,), jnp.float32),
            pltpu.VMEM((LANES,), jnp.int32),
            pltpu.VMEM((2 * LANES,), jnp.float32),
            pltpu.VMEM((2 * LANES * LANES,), jnp.float32),
            pltpu.VMEM_SHARED((2 * LANES * LANES,), jnp.float32),
            pltpu.VMEM((32 * LANES,), jnp.float32),
            pltpu.VMEM((32 * LANES,), jnp.float32),
            pltpu.VMEM((512 * LANES,), jnp.float32),
            pltpu.VMEM((16 * LANES,), jnp.float32),
            pltpu.VMEM((48 * LANES,), jnp.float32),
            pltpu.VMEM((3 * LANES,), jnp.float32),
        ],
        interpret=_INTERPRET,
    )
    mask, wflags = sc_router(current_uncertainty, params)

    wf = wflags.reshape(NW, LANES)[:, 0].reshape(NBLK, NW // NBLK)
    flags = functools.reduce(jnp.bitwise_or,
                             [wf[:, g] for g in range(NW // NBLK)])

    wc1, bc1, wc2, bc2 = pl.pallas_call(
        _fuse_body,
        out_shape=(jax.ShapeDtypeStruct((D, D), jnp.float32),
                   jax.ShapeDtypeStruct((1, D), jnp.float32),
                   jax.ShapeDtypeStruct((D, D), jnp.float32),
                   jax.ShapeDtypeStruct((1, D), jnp.float32)),
        interpret=_INTERPRET,
    )(W0, b0.reshape(1, D), W1, b1.reshape(1, D),
      W2, b2.reshape(1, D), W3, b3.reshape(1, D))

    grid_spec = pltpu.PrefetchScalarGridSpec(
        num_scalar_prefetch=1,
        grid=(NBLK,),
        in_specs=[
            pl.BlockSpec((BLK, D), lambda i, flags: (i, 0)),
            pl.BlockSpec((BLK, 1), lambda i, flags: (i, 0)),
            _full((D, D)), _full((1, D)),
            _full((D, D)), _full((1, D)),
            _full((D, D)), _full((1, D)),
        ],
        out_specs=pl.BlockSpec((BLK, D), lambda i, flags: (i, 0)),
    )
    out = pl.pallas_call(
        _chain_body,
        grid_spec=grid_spec,
        out_shape=jax.ShapeDtypeStruct((N, D), jnp.float32),
        interpret=_INTERPRET,
    )(flags, x, mask.reshape(N, 1),
      W0, b0.reshape(1, D), wc1, bc1, wc2, bc2)
    return out, mask
